# baseline (speedup 1.0000x reference)
import functools

import jax
import jax.numpy as jnp
from jax import lax
from jax.experimental import pallas as pl
from jax.experimental.pallas import tpu as pltpu
from jax.experimental.pallas import tpu_sc as plsc

_N = 10000
_NP = 10240
_E = 320000
_R = 8
_D = 128
_NB = 34
_NRP = _NP * _R

_NC = 2
_NS = 16
_NW = _NC * _NS
_EP = 327680
_EPW = _EP // _NW

_CC = 128
_CR = _EP // _CC
_CRW = _CR // _NW

_CH = 64
_RPW = _EPW // _CH
_RPS = _NP // _NS
_CPS = _NRP // _NS
_CNT_ROWS = _NRP // _D

_sc_mesh = plsc.VectorSubcoreMesh(core_axis_name="c", subcore_axis_name="s")


@functools.partial(
    pl.kernel,
    out_type=[
        jax.ShapeDtypeStruct((_NC * _NRP,), jnp.int32),
        jax.ShapeDtypeStruct((_CR, _CC), jnp.int32),
        jax.ShapeDtypeStruct((_CR, _CC), jnp.int32),
    ],
    mesh=_sc_mesh,
    scratch_types=[
        pltpu.VMEM_SHARED((_NRP,), jnp.int32),
        pltpu.VMEM((_CRW, _CC), jnp.int32),
        pltpu.VMEM((_CRW, _CC), jnp.int32),
        pltpu.VMEM((_CRW, _CC), jnp.int32),
        pltpu.VMEM((_CC,), jnp.int32),
        pltpu.VMEM((_CPS,), jnp.int32),
        pltpu.SemaphoreType.DMA,
    ],
)
def _sc_count(src_hbm, dst_hbm, rel_hbm, zero_hbm, ones_hbm,
              cnt_out, g_out, nk_out,
              cnt_sp, src2, dst2, rel2, onesb, cstage, sem_s):
    c = lax.axis_index("c")
    s = lax.axis_index("s")
    wid = s * _NC + c
    pltpu.sync_copy(zero_hbm.at[pl.ds(s * _CPS, _CPS)], cstage)
    pltpu.sync_copy(cstage, cnt_sp.at[pl.ds(s * _CPS, _CPS)])
    pltpu.sync_copy(ones_hbm, onesb)
    pltpu.sync_copy(src_hbm.at[pl.ds(wid * _CRW, _CRW)], src2)
    pltpu.sync_copy(dst_hbm.at[pl.ds(wid * _CRW, _CRW)], dst2)
    pltpu.sync_copy(rel_hbm.at[pl.ds(wid * _CRW, _CRW)], rel2)

    @pl.loop(0, _CRW)
    def _keys(j):
        for t in range(_CC // 16):
            sl = pl.ds(t * 16, 16)
            rv = rel2[j, sl]
            src2[j, sl] = rv * _NP + src2[j, sl]
            rel2[j, sl] = dst2[j, sl] * _R + rv

    plsc.subcore_barrier()

    @pl.loop(0, _CRW)
    def _fire(j):
        pltpu.async_copy(onesb, cnt_sp.at[rel2.at[j]], sem_s, add=True)

        @pl.when(j >= 8)
        def _():
            pltpu.make_async_copy(onesb, cnt_sp.at[rel2.at[j]], sem_s).wait()

    @pl.loop(0, 8)
    def _drain(j):
        pltpu.make_async_copy(onesb, cnt_sp.at[rel2.at[j]], sem_s).wait()

    pltpu.sync_copy(src2, g_out.at[pl.ds(wid * _CRW, _CRW)])
    pltpu.sync_copy(rel2, nk_out.at[pl.ds(wid * _CRW, _CRW)])

    plsc.subcore_barrier()
    pltpu.sync_copy(cnt_sp.at[pl.ds(s * _CPS, _CPS)], cstage)
    pltpu.sync_copy(cstage, cnt_out.at[pl.ds(c * _NRP + s * _CPS, _CPS)])


@functools.partial(
    pl.kernel,
    out_type=jax.ShapeDtypeStruct((_NC, _NP, _D), jnp.float32),
    mesh=_sc_mesh,
    scratch_types=(
        [pltpu.VMEM_SHARED((_NP, _D), jnp.float32)]
        + [pltpu.VMEM((_CH, _D), jnp.float32)] * 4
        + [pltpu.VMEM((_CH,), jnp.int32)] * 4
        + [pltpu.VMEM((_CH,), jnp.int32)] * 4
        + [pltpu.VMEM((_CH,), jnp.int32)] * 4
        + [pltpu.VMEM((_CH,), jnp.float32)] * 4
        + [pltpu.SemaphoreType.DMA] * 16
    ),
)
def _sc_layer(g_hbm, nk_hbm, dst_hbm, xw_hbm, inv_hbm, zero_hbm, acc_out,
              acc_sp, *bufs):
    rows = bufs[0:4]
    gb = bufs[4:8]
    nkb = bufs[8:12]
    dstb = bufs[12:16]
    invv = bufs[16:20]
    sg = bufs[20:24]
    si = bufs[24:28]
    ss = bufs[28:32]
    sx = bufs[32:36]
    c = lax.axis_index("c")
    s = lax.axis_index("s")
    wid = s * _NC + c
    ebase = wid * _EPW

    pltpu.sync_copy(zero_hbm.at[pl.ds(0, _CH)], rows[0])
    for t in range(_RPS // _CH):
        pltpu.sync_copy(rows[0], acc_sp.at[pl.ds(s * _RPS + t * _CH, _CH)])
    plsc.subcore_barrier()

    def start_idx(j, b):
        o = ebase + j * _CH
        pltpu.async_copy(g_hbm.at[pl.ds(o, _CH)], gb[b], sx[b])
        pltpu.async_copy(nk_hbm.at[pl.ds(o, _CH)], nkb[b], sx[b])
        pltpu.async_copy(dst_hbm.at[pl.ds(o, _CH)], dstb[b], sx[b])

    def wait_idx(j, b):
        o = ebase + j * _CH
        pltpu.make_async_copy(g_hbm.at[pl.ds(o, _CH)], gb[b], sx[b]).wait()
        pltpu.make_async_copy(nk_hbm.at[pl.ds(o, _CH)], nkb[b], sx[b]).wait()
        pltpu.make_async_copy(dst_hbm.at[pl.ds(o, _CH)], dstb[b], sx[b]).wait()

    def start_g(b):
        pltpu.async_copy(xw_hbm.at[gb[b]], rows[b], sg[b])
        pltpu.async_copy(inv_hbm.at[nkb[b]], invv[b], si[b])

    def wait_g(b):
        pltpu.make_async_copy(xw_hbm.at[gb[b]], rows[b], sg[b]).wait()
        pltpu.make_async_copy(inv_hbm.at[nkb[b]], invv[b], si[b]).wait()

    def start_s(b):
        pltpu.async_copy(rows[b], acc_sp.at[dstb[b]], ss[b], add=True)

    def wait_s(b):
        pltpu.make_async_copy(rows[b], acc_sp.at[dstb[b]], ss[b]).wait()

    def scale(b):
        @pl.loop(0, _CH // 16)
        def _scale(i16):
            i0 = i16 * 16
            iv = invv[b][pl.ds(i0, 16)]
            for t in range(16):
                sv = jnp.full((16,), iv[t])
                for u in range(_D // 16):
                    sl = pl.ds(u * 16, 16)
                    rows[b][i0 + t, sl] = rows[b][i0 + t, sl] * sv

    start_idx(0, 0)
    start_idx(1, 1)
    wait_idx(0, 0)
    start_g(0)
    for j in range(2):
        b = j % 4
        wait_g(b)
        wait_idx(j + 1, (b + 1) % 4)
        start_g((b + 1) % 4)
        start_idx(j + 2, (b + 2) % 4)
        scale(b)
        start_s(b)

    @pl.loop(2, _RPW + 2, step=4)
    def _main(jj):
        for t in range(4):
            j = jj + t
            b = (2 + t) % 4

            @pl.when(j <= _RPW - 1)
            def _():
                wait_g(b)
                wait_s((b + 2) % 4)

                @pl.when(j <= _RPW - 2)
                def _():
                    wait_idx(j + 1, (b + 1) % 4)
                    start_g((b + 1) % 4)

                @pl.when(j <= _RPW - 3)
                def _():
                    start_idx(j + 2, (b + 2) % 4)

                scale(b)
                start_s(b)

    wait_s((_RPW - 2) % 4)
    wait_s((_RPW - 1) % 4)

    plsc.subcore_barrier()
    for t in range(_RPS // _CH):
        pltpu.sync_copy(acc_sp.at[pl.ds(s * _RPS + t * _CH, _CH)], rows[0])
        pltpu.sync_copy(rows[0], acc_out.at[c, pl.ds(s * _RPS + t * _CH, _CH)])


def _wmix_body(comp_ref, bases_ref, w_ref):
    w_ref[...] = jnp.dot(comp_ref[...], bases_ref[...],
                         preferred_element_type=jnp.float32)


def _wmix(comp, bases2d):
    return pl.pallas_call(
        _wmix_body,
        out_shape=jax.ShapeDtypeStruct((_R, _D * _D), jnp.float32),
    )(comp, bases2d)


def _inv_body(cnt_ref, inv_ref):
    ctot = cnt_ref[0] + cnt_ref[1]
    inv_ref[...] = 1.0 / jnp.maximum(ctot, 1).astype(jnp.float32)


def _invk(cnt):
    return pl.pallas_call(
        _inv_body,
        out_shape=jax.ShapeDtypeStruct((_CNT_ROWS, _D), jnp.float32),
    )(cnt)


_BN = 640
_NBLK = _NP // _BN


def _dense1_body(x_ref, w_ref, root_ref, bias_ref, xw_ref, rootx_ref):
    xb = x_ref[...]
    for r in range(_R):
        xw_ref[r] = jnp.dot(xb, w_ref[r], preferred_element_type=jnp.float32)
    rootx_ref[...] = (jnp.dot(xb, root_ref[...],
                              preferred_element_type=jnp.float32)
                      + bias_ref[...])


def _dense1(x, w, root, bias):
    return pl.pallas_call(
        _dense1_body,
        grid=(_NBLK,),
        in_specs=[
            pl.BlockSpec((_BN, _D), lambda i: (i, 0)),
            pl.BlockSpec((_R, _D, _D), lambda i: (0, 0, 0)),
            pl.BlockSpec((_D, _D), lambda i: (0, 0)),
            pl.BlockSpec((1, _D), lambda i: (0, 0)),
        ],
        out_specs=[
            pl.BlockSpec((_R, _BN, _D), lambda i: (0, i, 0)),
            pl.BlockSpec((_BN, _D), lambda i: (i, 0)),
        ],
        out_shape=[
            jax.ShapeDtypeStruct((_R, _NP, _D), jnp.float32),
            jax.ShapeDtypeStruct((_NP, _D), jnp.float32),
        ],
    )(x, w, root, bias)


def _dense2_body(acc_ref, rootx1_ref, w_ref, root_ref, bias_ref,
                 xw_ref, rootx_ref):
    xb = acc_ref[0] + acc_ref[1] + rootx1_ref[...]
    for r in range(_R):
        xw_ref[r] = jnp.dot(xb, w_ref[r], preferred_element_type=jnp.float32)
    rootx_ref[...] = (jnp.dot(xb, root_ref[...],
                              preferred_element_type=jnp.float32)
                      + bias_ref[...])


def _dense2(acc, rootx1, w, root, bias):
    return pl.pallas_call(
        _dense2_body,
        grid=(_NBLK,),
        in_specs=[
            pl.BlockSpec((_NC, _BN, _D), lambda i: (0, i, 0)),
            pl.BlockSpec((_BN, _D), lambda i: (i, 0)),
            pl.BlockSpec((_R, _D, _D), lambda i: (0, 0, 0)),
            pl.BlockSpec((_D, _D), lambda i: (0, 0)),
            pl.BlockSpec((1, _D), lambda i: (0, 0)),
        ],
        out_specs=[
            pl.BlockSpec((_R, _BN, _D), lambda i: (0, i, 0)),
            pl.BlockSpec((_BN, _D), lambda i: (i, 0)),
        ],
        out_shape=[
            jax.ShapeDtypeStruct((_R, _NP, _D), jnp.float32),
            jax.ShapeDtypeStruct((_NP, _D), jnp.float32),
        ],
    )(acc, rootx1, w, root, bias)


def _final_body(acc_ref, rootx_ref, out_ref):
    out_ref[...] = acc_ref[0] + acc_ref[1] + rootx_ref[...]


def _final(acc, rootx):
    return pl.pallas_call(
        _final_body,
        grid=(_NBLK,),
        in_specs=[
            pl.BlockSpec((_NC, _BN, _D), lambda i: (0, i, 0)),
            pl.BlockSpec((_BN, _D), lambda i: (i, 0)),
        ],
        out_specs=pl.BlockSpec((_BN, _D), lambda i: (i, 0)),
        out_shape=jax.ShapeDtypeStruct((_NP, _D), jnp.float32),
    )(acc, rootx)


def kernel(node_index, edge_index, edge_type, node_frequency, node_emb,
           comp1, bases1, root1, bias1, comp2, bases2, root2, bias2):
    del node_frequency
    x = node_emb[node_index]
    x = jnp.pad(x, ((0, _NP - _N), (0, 0)))
    pad = _EP - _E
    srcs = jnp.concatenate(
        [edge_index[0].astype(jnp.int32), jnp.zeros((pad,), jnp.int32)]
    ).reshape(_CR, _CC)
    dsts = jnp.concatenate(
        [edge_index[1].astype(jnp.int32), jnp.full((pad,), _N, jnp.int32)]
    ).reshape(_CR, _CC)
    rels = jnp.concatenate(
        [edge_type.astype(jnp.int32), jnp.zeros((pad,), jnp.int32)]
    ).reshape(_CR, _CC)
    zero_f = jnp.zeros((_NP, _D), jnp.float32)
    zero_i = jnp.zeros((_NRP,), jnp.int32)
    ones_i = jnp.ones((_CC,), jnp.int32)

    cnt, g2d, nk2d = _sc_count(srcs, dsts, rels, zero_i, ones_i)
    g_flat = g2d.reshape(_EP)
    nk_flat = nk2d.reshape(_EP)
    dst_flat = dsts.reshape(_EP)
    w1 = _wmix(comp1, bases1.reshape(_NB, _D * _D)).reshape(_R, _D, _D)
    w2 = _wmix(comp2, bases2.reshape(_NB, _D * _D)).reshape(_R, _D, _D)
    inv_flat = _invk(cnt.reshape(_NC, _CNT_ROWS, _D)).reshape(_NRP)

    xw1, rootx1 = _dense1(x, w1, root1, bias1.reshape(1, _D))
    acc1 = _sc_layer(g_flat, nk_flat, dst_flat, xw1.reshape(_R * _NP, _D),
                     inv_flat, zero_f)
    xw2, rootx2 = _dense2(acc1, rootx1, w2, root2, bias2.reshape(1, _D))
    acc2 = _sc_layer(g_flat, nk_flat, dst_flat, xw2.reshape(_R * _NP, _D),
                     inv_flat, zero_f)
    return _final(acc2, rootx2)[:_N]


# speedup vs baseline: 2.6087x; 2.6087x over previous
import functools

import jax
import jax.numpy as jnp
from jax import lax
from jax.experimental import pallas as pl
from jax.experimental.pallas import tpu as pltpu
from jax.experimental.pallas import tpu_sc as plsc

_N = 10000
_NP = 10240
_E = 320000
_R = 8
_D = 128
_NB = 34
_NRP = _NP * _R

_NC = 2
_NS = 16
_NW = _NC * _NS
_EP = 327680
_EPW = _EP // _NW

_CC = 128
_CR = _EP // _CC
_CRW = _CR // _NW

_CH = 64
_RPW = _EPW // _CH
_RPS = _NP // _NS
_CPS = _NRP // _NS
_CNT_ROWS = _NRP // _D

_sc_mesh = plsc.VectorSubcoreMesh(core_axis_name="c", subcore_axis_name="s")


@functools.partial(
    pl.kernel,
    out_type=[
        jax.ShapeDtypeStruct((_NC * _NRP,), jnp.int32),
        jax.ShapeDtypeStruct((_CR, _CC), jnp.int32),
        jax.ShapeDtypeStruct((_CR, _CC), jnp.int32),
    ],
    mesh=_sc_mesh,
    scratch_types=[
        pltpu.VMEM_SHARED((_NRP,), jnp.int32),
        pltpu.VMEM((_CRW, _CC), jnp.int32),
        pltpu.VMEM((_CRW, _CC), jnp.int32),
        pltpu.VMEM((_CRW, _CC), jnp.int32),
        pltpu.VMEM((_CC,), jnp.int32),
        pltpu.VMEM((_CPS,), jnp.int32),
        pltpu.SemaphoreType.DMA,
    ],
)
def _sc_count(src_hbm, dst_hbm, rel_hbm, zero_hbm, ones_hbm,
              cnt_out, g_out, nk_out,
              cnt_sp, src2, dst2, rel2, onesb, cstage, sem_s):
    c = lax.axis_index("c")
    s = lax.axis_index("s")
    wid = s * _NC + c
    pltpu.sync_copy(zero_hbm.at[pl.ds(s * _CPS, _CPS)], cstage)
    pltpu.sync_copy(cstage, cnt_sp.at[pl.ds(s * _CPS, _CPS)])
    pltpu.sync_copy(ones_hbm, onesb)
    pltpu.sync_copy(src_hbm.at[pl.ds(wid * _CRW, _CRW)], src2)
    pltpu.sync_copy(dst_hbm.at[pl.ds(wid * _CRW, _CRW)], dst2)
    pltpu.sync_copy(rel_hbm.at[pl.ds(wid * _CRW, _CRW)], rel2)

    @pl.loop(0, _CRW)
    def _keys(j):
        for t in range(_CC // 16):
            sl = pl.ds(t * 16, 16)
            rv = rel2[j, sl]
            src2[j, sl] = rv * _NP + src2[j, sl]
            rel2[j, sl] = dst2[j, sl] * _R + rv

    plsc.subcore_barrier()

    @pl.loop(0, _CRW)
    def _fire(j):
        pltpu.async_copy(onesb, cnt_sp.at[rel2.at[j]], sem_s, add=True)

        @pl.when(j >= 8)
        def _():
            pltpu.make_async_copy(onesb, cnt_sp.at[rel2.at[j]], sem_s).wait()

    @pl.loop(0, 8)
    def _drain(j):
        pltpu.make_async_copy(onesb, cnt_sp.at[rel2.at[j]], sem_s).wait()

    pltpu.sync_copy(src2, g_out.at[pl.ds(wid * _CRW, _CRW)])
    pltpu.sync_copy(rel2, nk_out.at[pl.ds(wid * _CRW, _CRW)])

    plsc.subcore_barrier()
    pltpu.sync_copy(cnt_sp.at[pl.ds(s * _CPS, _CPS)], cstage)
    pltpu.sync_copy(cstage, cnt_out.at[pl.ds(c * _NRP + s * _CPS, _CPS)])


@functools.partial(
    pl.kernel,
    out_type=jax.ShapeDtypeStruct((_NC, _NP, _D), jnp.float32),
    mesh=_sc_mesh,
    scratch_types=(
        [pltpu.VMEM_SHARED((_NP, _D), jnp.float32)]
        + [pltpu.VMEM((_CH, _D), jnp.float32)] * 4
        + [pltpu.VMEM((_CH,), jnp.int32)] * 4
        + [pltpu.VMEM((_CH,), jnp.int32)] * 4
        + [pltpu.VMEM((_CH,), jnp.int32)] * 4
        + [pltpu.VMEM((_CH,), jnp.float32)] * 4
        + [pltpu.SemaphoreType.DMA] * 16
    ),
)
def _sc_layer(g_hbm, nk_hbm, dst_hbm, xw_hbm, inv_hbm, zero_hbm, acc_out,
              acc_sp, *bufs):
    rows = bufs[0:4]
    gb = bufs[4:8]
    nkb = bufs[8:12]
    dstb = bufs[12:16]
    invv = bufs[16:20]
    sg = bufs[20:24]
    si = bufs[24:28]
    ss = bufs[28:32]
    sx = bufs[32:36]
    c = lax.axis_index("c")
    s = lax.axis_index("s")
    wid = s * _NC + c
    ebase = wid * _EPW

    pltpu.sync_copy(zero_hbm.at[pl.ds(0, _CH)], rows[0])
    for t in range(_RPS // _CH):
        pltpu.sync_copy(rows[0], acc_sp.at[pl.ds(s * _RPS + t * _CH, _CH)])
    plsc.subcore_barrier()

    def start_idx(j, b):
        o = ebase + j * _CH
        pltpu.async_copy(g_hbm.at[pl.ds(o, _CH)], gb[b], sx[b])
        pltpu.async_copy(nk_hbm.at[pl.ds(o, _CH)], nkb[b], sx[b])
        pltpu.async_copy(dst_hbm.at[pl.ds(o, _CH)], dstb[b], sx[b])

    def wait_idx(j, b):
        o = ebase + j * _CH
        pltpu.make_async_copy(g_hbm.at[pl.ds(o, _CH)], gb[b], sx[b]).wait()
        pltpu.make_async_copy(nk_hbm.at[pl.ds(o, _CH)], nkb[b], sx[b]).wait()
        pltpu.make_async_copy(dst_hbm.at[pl.ds(o, _CH)], dstb[b], sx[b]).wait()

    def start_g(b):
        pltpu.async_copy(xw_hbm.at[gb[b]], rows[b], sg[b])
        pltpu.async_copy(inv_hbm.at[nkb[b]], invv[b], si[b])

    def wait_g(b):
        pltpu.make_async_copy(xw_hbm.at[gb[b]], rows[b], sg[b]).wait()
        pltpu.make_async_copy(inv_hbm.at[nkb[b]], invv[b], si[b]).wait()

    def start_s(b):
        pltpu.async_copy(rows[b], acc_sp.at[dstb[b]], ss[b], add=True)

    def wait_s(b):
        pltpu.make_async_copy(rows[b], acc_sp.at[dstb[b]], ss[b]).wait()

    def scale(b):
        @pl.loop(0, _CH // 16)
        def _scale(i16):
            i0 = i16 * 16
            iv = invv[b][pl.ds(i0, 16)]
            for t in range(16):
                sv = jnp.full((16,), iv[t])
                for u in range(_D // 16):
                    sl = pl.ds(u * 16, 16)
                    rows[b][i0 + t, sl] = rows[b][i0 + t, sl] * sv

    start_idx(0, 0)
    start_idx(1, 1)
    wait_idx(0, 0)
    start_g(0)
    for j in range(2):
        b = j % 4
        wait_g(b)
        wait_idx(j + 1, (b + 1) % 4)
        start_g((b + 1) % 4)
        start_idx(j + 2, (b + 2) % 4)
        scale(b)
        start_s(b)

    @pl.loop(2, _RPW + 2, step=4)
    def _main(jj):
        for t in range(4):
            j = jj + t
            b = (2 + t) % 4

            @pl.when(j <= _RPW - 1)
            def _():
                wait_g(b)
                wait_s((b + 2) % 4)

                @pl.when(j <= _RPW - 2)
                def _():
                    wait_idx(j + 1, (b + 1) % 4)
                    start_g((b + 1) % 4)

                @pl.when(j <= _RPW - 3)
                def _():
                    start_idx(j + 2, (b + 2) % 4)

                scale(b)
                start_s(b)

    wait_s((_RPW - 2) % 4)
    wait_s((_RPW - 1) % 4)

    plsc.subcore_barrier()
    for t in range(_RPS // _CH):
        pltpu.sync_copy(acc_sp.at[pl.ds(s * _RPS + t * _CH, _CH)], rows[0])
        pltpu.sync_copy(rows[0], acc_out.at[c, pl.ds(s * _RPS + t * _CH, _CH)])


def _wmix_body(comp_ref, bases_ref, w_ref):
    w_ref[...] = jnp.dot(comp_ref[...], bases_ref[...],
                         preferred_element_type=jnp.float32)


def _wmix(comp, bases2d):
    return pl.pallas_call(
        _wmix_body,
        out_shape=jax.ShapeDtypeStruct((_R, _D * _D), jnp.float32),
    )(comp, bases2d)


def _inv_body(cnt_ref, inv_ref):
    ctot = cnt_ref[0] + cnt_ref[1]
    inv_ref[...] = 1.0 / jnp.maximum(ctot, 1).astype(jnp.float32)


def _invk(cnt):
    return pl.pallas_call(
        _inv_body,
        out_shape=jax.ShapeDtypeStruct((_CNT_ROWS, _D), jnp.float32),
    )(cnt)


_BN = 640
_NBLK = _NP // _BN


def _dense1_body(x_ref, w_ref, root_ref, bias_ref, xw_ref, rootx_ref):
    xb = x_ref[...]
    for r in range(_R):
        xw_ref[r] = jnp.dot(xb, w_ref[r], preferred_element_type=jnp.float32)
    rootx_ref[...] = (jnp.dot(xb, root_ref[...],
                              preferred_element_type=jnp.float32)
                      + bias_ref[...])


def _dense1(x, w, root, bias):
    return pl.pallas_call(
        _dense1_body,
        grid=(_NBLK,),
        in_specs=[
            pl.BlockSpec((_BN, _D), lambda i: (i, 0)),
            pl.BlockSpec((_R, _D, _D), lambda i: (0, 0, 0)),
            pl.BlockSpec((_D, _D), lambda i: (0, 0)),
            pl.BlockSpec((1, _D), lambda i: (0, 0)),
        ],
        out_specs=[
            pl.BlockSpec((_R, _BN, _D), lambda i: (0, i, 0)),
            pl.BlockSpec((_BN, _D), lambda i: (i, 0)),
        ],
        out_shape=[
            jax.ShapeDtypeStruct((_R, _NP, _D), jnp.float32),
            jax.ShapeDtypeStruct((_NP, _D), jnp.float32),
        ],
    )(x, w, root, bias)


def _dense2_body(acc_ref, rootx1_ref, w_ref, root_ref, bias_ref,
                 xw_ref, rootx_ref):
    xb = acc_ref[0] + acc_ref[1] + rootx1_ref[...]
    for r in range(_R):
        xw_ref[r] = jnp.dot(xb, w_ref[r], preferred_element_type=jnp.float32)
    rootx_ref[...] = (jnp.dot(xb, root_ref[...],
                              preferred_element_type=jnp.float32)
                      + bias_ref[...])


def _dense2(acc, rootx1, w, root, bias):
    return pl.pallas_call(
        _dense2_body,
        grid=(_NBLK,),
        in_specs=[
            pl.BlockSpec((_NC, _BN, _D), lambda i: (0, i, 0)),
            pl.BlockSpec((_BN, _D), lambda i: (i, 0)),
            pl.BlockSpec((_R, _D, _D), lambda i: (0, 0, 0)),
            pl.BlockSpec((_D, _D), lambda i: (0, 0)),
            pl.BlockSpec((1, _D), lambda i: (0, 0)),
        ],
        out_specs=[
            pl.BlockSpec((_R, _BN, _D), lambda i: (0, i, 0)),
            pl.BlockSpec((_BN, _D), lambda i: (i, 0)),
        ],
        out_shape=[
            jax.ShapeDtypeStruct((_R, _NP, _D), jnp.float32),
            jax.ShapeDtypeStruct((_NP, _D), jnp.float32),
        ],
    )(acc, rootx1, w, root, bias)


def _final_body(acc_ref, rootx_ref, out_ref):
    out_ref[...] = acc_ref[0] + acc_ref[1] + rootx_ref[...]


def _final(acc, rootx):
    return pl.pallas_call(
        _final_body,
        grid=(_NBLK,),
        in_specs=[
            pl.BlockSpec((_NC, _BN, _D), lambda i: (0, i, 0)),
            pl.BlockSpec((_BN, _D), lambda i: (i, 0)),
        ],
        out_specs=pl.BlockSpec((_BN, _D), lambda i: (i, 0)),
        out_shape=jax.ShapeDtypeStruct((_NP, _D), jnp.float32),
    )(acc, rootx)


def kernel(node_index, edge_index, edge_type, node_frequency, node_emb,
           comp1, bases1, root1, bias1, comp2, bases2, root2, bias2):
    del node_frequency
    x = node_emb[node_index]
    x = jnp.pad(x, ((0, _NP - _N), (0, 0)))
    pad = _EP - _E
    pad_iota = jnp.arange(pad, dtype=jnp.int32)
    srcs = jnp.concatenate(
        [edge_index[0].astype(jnp.int32), pad_iota % _N]
    ).reshape(_CR, _CC)
    dsts = jnp.concatenate(
        [edge_index[1].astype(jnp.int32), _N + pad_iota % (_NP - _N)]
    ).reshape(_CR, _CC)
    rels = jnp.concatenate(
        [edge_type.astype(jnp.int32), jnp.zeros((pad,), jnp.int32)]
    ).reshape(_CR, _CC)
    zero_f = jnp.zeros((_NP, _D), jnp.float32)
    zero_i = jnp.zeros((_NRP,), jnp.int32)
    ones_i = jnp.ones((_CC,), jnp.int32)

    cnt, g2d, nk2d = _sc_count(srcs, dsts, rels, zero_i, ones_i)
    g_flat = g2d.reshape(_EP)
    nk_flat = nk2d.reshape(_EP)
    dst_flat = dsts.reshape(_EP)
    w1 = _wmix(comp1, bases1.reshape(_NB, _D * _D)).reshape(_R, _D, _D)
    w2 = _wmix(comp2, bases2.reshape(_NB, _D * _D)).reshape(_R, _D, _D)
    inv_flat = _invk(cnt.reshape(_NC, _CNT_ROWS, _D)).reshape(_NRP)

    xw1, rootx1 = _dense1(x, w1, root1, bias1.reshape(1, _D))
    acc1 = _sc_layer(g_flat, nk_flat, dst_flat, xw1.reshape(_R * _NP, _D),
                     inv_flat, zero_f)
    xw2, rootx2 = _dense2(acc1, rootx1, w2, root2, bias2.reshape(1, _D))
    acc2 = _sc_layer(g_flat, nk_flat, dst_flat, xw2.reshape(_R * _NP, _D),
                     inv_flat, zero_f)
    return _final(acc2, rootx2)[:_N]


# speedup vs baseline: 2.6109x; 1.0008x over previous
import functools

import jax
import jax.numpy as jnp
from jax import lax
from jax.experimental import pallas as pl
from jax.experimental.pallas import tpu as pltpu
from jax.experimental.pallas import tpu_sc as plsc

_N = 10000
_NP = 10240
_E = 320000
_R = 8
_D = 128
_NB = 34
_NRP = _NP * _R

_NC = 2
_NS = 16
_NW = _NC * _NS
_EP = 327680
_EPW = _EP // _NW

_CC = 128
_CR = _EP // _CC
_CRW = _CR // _NW

_CH = 64
_RPW = _EPW // _CH
_RPS = _NP // _NS
_CPS = _NRP // _NS
_CNT_ROWS = _NRP // _D

_sc_mesh = plsc.VectorSubcoreMesh(core_axis_name="c", subcore_axis_name="s")


@functools.partial(
    pl.kernel,
    out_type=[
        jax.ShapeDtypeStruct((_NC * _NRP,), jnp.int32),
        jax.ShapeDtypeStruct((_CR, _CC), jnp.int32),
        jax.ShapeDtypeStruct((_CR, _CC), jnp.int32),
    ],
    mesh=_sc_mesh,
    scratch_types=[
        pltpu.VMEM_SHARED((_NRP,), jnp.int32),
        pltpu.VMEM((_CRW, _CC), jnp.int32),
        pltpu.VMEM((_CRW, _CC), jnp.int32),
        pltpu.VMEM((_CRW, _CC), jnp.int32),
        pltpu.VMEM((_CC,), jnp.int32),
        pltpu.VMEM((_CPS,), jnp.int32),
        pltpu.SemaphoreType.DMA,
    ],
)
def _sc_count(src_hbm, dst_hbm, rel_hbm, zero_hbm, ones_hbm,
              cnt_out, g_out, nk_out,
              cnt_sp, src2, dst2, rel2, onesb, cstage, sem_s):
    c = lax.axis_index("c")
    s = lax.axis_index("s")
    wid = s * _NC + c
    pltpu.sync_copy(zero_hbm.at[pl.ds(s * _CPS, _CPS)], cstage)
    pltpu.sync_copy(cstage, cnt_sp.at[pl.ds(s * _CPS, _CPS)])
    pltpu.sync_copy(ones_hbm, onesb)
    pltpu.sync_copy(src_hbm.at[pl.ds(wid * _CRW, _CRW)], src2)
    pltpu.sync_copy(dst_hbm.at[pl.ds(wid * _CRW, _CRW)], dst2)
    pltpu.sync_copy(rel_hbm.at[pl.ds(wid * _CRW, _CRW)], rel2)

    @pl.loop(0, _CRW)
    def _keys(j):
        for t in range(_CC // 16):
            sl = pl.ds(t * 16, 16)
            rv = rel2[j, sl]
            src2[j, sl] = rv * _NP + src2[j, sl]
            rel2[j, sl] = dst2[j, sl] * _R + rv

    plsc.subcore_barrier()

    @pl.loop(0, _CRW)
    def _fire(j):
        pltpu.async_copy(onesb, cnt_sp.at[rel2.at[j]], sem_s, add=True)

        @pl.when(j >= 8)
        def _():
            pltpu.make_async_copy(onesb, cnt_sp.at[rel2.at[j]], sem_s).wait()

    @pl.loop(0, 8)
    def _drain(j):
        pltpu.make_async_copy(onesb, cnt_sp.at[rel2.at[j]], sem_s).wait()

    pltpu.sync_copy(src2, g_out.at[pl.ds(wid * _CRW, _CRW)])
    pltpu.sync_copy(rel2, nk_out.at[pl.ds(wid * _CRW, _CRW)])

    plsc.subcore_barrier()
    pltpu.sync_copy(cnt_sp.at[pl.ds(s * _CPS, _CPS)], cstage)
    pltpu.sync_copy(cstage, cnt_out.at[pl.ds(c * _NRP + s * _CPS, _CPS)])


@functools.partial(
    pl.kernel,
    out_type=jax.ShapeDtypeStruct((_NC, _NP, _D), jnp.float32),
    mesh=_sc_mesh,
    scratch_types=(
        [pltpu.VMEM_SHARED((_NP, _D), jnp.float32)]
        + [pltpu.VMEM((_CH, _D), jnp.float32)] * 4
        + [pltpu.VMEM((_CH,), jnp.int32)] * 2
        + [pltpu.VMEM((_CH,), jnp.int32)] * 2
        + [pltpu.VMEM((_CH,), jnp.int32)] * 4
        + [pltpu.VMEM((_CH,), jnp.float32)] * 2
        + [pltpu.SemaphoreType.DMA] * 12
    ),
)
def _sc_layer(g_hbm, nk_hbm, dst_hbm, xw_hbm, inv_hbm, zero_hbm, acc_out,
              acc_sp, *bufs):
    rows = bufs[0:4]
    gb = bufs[4:6]
    nkb = bufs[6:8]
    dstb = bufs[8:12]
    invv = bufs[12:14]
    sg = bufs[14:18]
    si = bufs[18:20]
    sx = bufs[20:22]
    ss = bufs[22:26]
    c = lax.axis_index("c")
    s = lax.axis_index("s")
    wid = s * _NC + c
    ebase = wid * _EPW

    pltpu.sync_copy(zero_hbm, rows[0])
    for t in range(_RPS // _CH):
        pltpu.sync_copy(rows[0], acc_sp.at[pl.ds(s * _RPS + t * _CH, _CH)])
    plsc.subcore_barrier()

    def start_idx(j, p, q):
        o = ebase + j * _CH
        pltpu.async_copy(g_hbm.at[pl.ds(o, _CH)], gb[p], sx[p])
        pltpu.async_copy(nk_hbm.at[pl.ds(o, _CH)], nkb[p], sx[p])
        pltpu.async_copy(dst_hbm.at[pl.ds(o, _CH)], dstb[q], sx[p])

    def wait_idx(j, p, q):
        o = ebase + j * _CH
        pltpu.make_async_copy(g_hbm.at[pl.ds(o, _CH)], gb[p], sx[p]).wait()
        pltpu.make_async_copy(nk_hbm.at[pl.ds(o, _CH)], nkb[p], sx[p]).wait()
        pltpu.make_async_copy(
            dst_hbm.at[pl.ds(o, _CH)], dstb[q], sx[p]).wait()

    def start_g(b, p):
        pltpu.async_copy(xw_hbm.at[gb[p]], rows[b], sg[b])
        pltpu.async_copy(inv_hbm.at[nkb[p]], invv[p], si[p])

    def wait_g(b, p):
        pltpu.make_async_copy(xw_hbm.at[gb[p]], rows[b], sg[b]).wait()
        pltpu.make_async_copy(inv_hbm.at[nkb[p]], invv[p], si[p]).wait()

    def start_s(b, q):
        pltpu.async_copy(rows[b], acc_sp.at[dstb[q]], ss[b], add=True)

    def wait_s(b, q):
        pltpu.make_async_copy(rows[b], acc_sp.at[dstb[q]], ss[b]).wait()

    def scale(b, p):
        @pl.loop(0, _CH // 16)
        def _scale(i16):
            i0 = i16 * 16
            iv = invv[p][pl.ds(i0, 16)]
            for t in range(16):
                sv = jnp.full((16,), iv[t])
                for u in range(_D // 16):
                    sl = pl.ds(u * 16, 16)
                    rows[b][i0 + t, sl] = rows[b][i0 + t, sl] * sv

    start_idx(0, 0, 0)
    start_idx(1, 1, 1)
    wait_idx(0, 0, 0)
    start_g(0, 0)
    for j in range(2):
        wait_g(j % 4, j % 2)
        wait_idx(j + 1, (j + 1) % 2, (j + 1) % 4)
        start_g((j + 1) % 4, (j + 1) % 2)
        start_idx(j + 2, (j + 2) % 2, (j + 2) % 4)
        scale(j % 4, j % 2)
        start_s(j % 4, j % 4)

    @pl.loop(2, _RPW + 2, step=4)
    def _main(jj):
        for t in range(4):
            j = jj + t
            p = t % 2
            b = (2 + t) % 4

            @pl.when(j <= _RPW - 1)
            def _():
                wait_g(b, p)
                wait_s((b + 2) % 4, (b + 2) % 4)

                @pl.when(j <= _RPW - 2)
                def _():
                    wait_idx(j + 1, (p + 1) % 2, (b + 1) % 4)
                    start_g((b + 1) % 4, (p + 1) % 2)

                @pl.when(j <= _RPW - 3)
                def _():
                    start_idx(j + 2, p, (b + 2) % 4)

                scale(b, p)
                start_s(b, b)

    wait_s((_RPW - 2) % 4, (_RPW - 2) % 4)
    wait_s((_RPW - 1) % 4, (_RPW - 1) % 4)

    plsc.subcore_barrier()
    for t in range(_RPS // _CH):
        pltpu.sync_copy(acc_sp.at[pl.ds(s * _RPS + t * _CH, _CH)], rows[0])
        pltpu.sync_copy(rows[0],
                        acc_out.at[c, pl.ds(s * _RPS + t * _CH, _CH)])


def _wmix_body(comp_ref, bases_ref, w_ref):
    w_ref[...] = jnp.dot(comp_ref[...], bases_ref[...],
                         preferred_element_type=jnp.float32)


def _wmix(comp, bases2d):
    return pl.pallas_call(
        _wmix_body,
        out_shape=jax.ShapeDtypeStruct((_R, _D * _D), jnp.float32),
    )(comp, bases2d)


def _inv_body(cnt_ref, inv_ref):
    ctot = cnt_ref[0] + cnt_ref[1]
    inv_ref[...] = 1.0 / jnp.maximum(ctot, 1).astype(jnp.float32)


def _invk(cnt):
    return pl.pallas_call(
        _inv_body,
        out_shape=jax.ShapeDtypeStruct((_CNT_ROWS, _D), jnp.float32),
    )(cnt)


_BN = 640
_NBLK = _NP // _BN


def _dense1_body(x_ref, w_ref, root_ref, bias_ref, xw_ref, rootx_ref):
    xb = x_ref[...]
    for r in range(_R):
        xw_ref[r] = jnp.dot(xb, w_ref[r], preferred_element_type=jnp.float32)
    rootx_ref[...] = (jnp.dot(xb, root_ref[...],
                              preferred_element_type=jnp.float32)
                      + bias_ref[...])


def _dense1(x, w, root, bias):
    return pl.pallas_call(
        _dense1_body,
        grid=(_NBLK,),
        in_specs=[
            pl.BlockSpec((_BN, _D), lambda i: (i, 0)),
            pl.BlockSpec((_R, _D, _D), lambda i: (0, 0, 0)),
            pl.BlockSpec((_D, _D), lambda i: (0, 0)),
            pl.BlockSpec((1, _D), lambda i: (0, 0)),
        ],
        out_specs=[
            pl.BlockSpec((_R, _BN, _D), lambda i: (0, i, 0)),
            pl.BlockSpec((_BN, _D), lambda i: (i, 0)),
        ],
        out_shape=[
            jax.ShapeDtypeStruct((_R, _NP, _D), jnp.float32),
            jax.ShapeDtypeStruct((_NP, _D), jnp.float32),
        ],
    )(x, w, root, bias)


def _dense2_body(acc_ref, rootx1_ref, w_ref, root_ref, bias_ref,
                 xw_ref, rootx_ref):
    xb = acc_ref[0] + acc_ref[1] + rootx1_ref[...]
    for r in range(_R):
        xw_ref[r] = jnp.dot(xb, w_ref[r], preferred_element_type=jnp.float32)
    rootx_ref[...] = (jnp.dot(xb, root_ref[...],
                              preferred_element_type=jnp.float32)
                      + bias_ref[...])


def _dense2(acc, rootx1, w, root, bias):
    return pl.pallas_call(
        _dense2_body,
        grid=(_NBLK,),
        in_specs=[
            pl.BlockSpec((_NC, _BN, _D), lambda i: (0, i, 0)),
            pl.BlockSpec((_BN, _D), lambda i: (i, 0)),
            pl.BlockSpec((_R, _D, _D), lambda i: (0, 0, 0)),
            pl.BlockSpec((_D, _D), lambda i: (0, 0)),
            pl.BlockSpec((1, _D), lambda i: (0, 0)),
        ],
        out_specs=[
            pl.BlockSpec((_R, _BN, _D), lambda i: (0, i, 0)),
            pl.BlockSpec((_BN, _D), lambda i: (i, 0)),
        ],
        out_shape=[
            jax.ShapeDtypeStruct((_R, _NP, _D), jnp.float32),
            jax.ShapeDtypeStruct((_NP, _D), jnp.float32),
        ],
    )(acc, rootx1, w, root, bias)


def _final_body(acc_ref, rootx_ref, out_ref):
    out_ref[...] = acc_ref[0] + acc_ref[1] + rootx_ref[...]


def _final(acc, rootx):
    return pl.pallas_call(
        _final_body,
        grid=(_NBLK,),
        in_specs=[
            pl.BlockSpec((_NC, _BN, _D), lambda i: (0, i, 0)),
            pl.BlockSpec((_BN, _D), lambda i: (i, 0)),
        ],
        out_specs=pl.BlockSpec((_BN, _D), lambda i: (i, 0)),
        out_shape=jax.ShapeDtypeStruct((_NP, _D), jnp.float32),
    )(acc, rootx)


def kernel(node_index, edge_index, edge_type, node_frequency, node_emb,
           comp1, bases1, root1, bias1, comp2, bases2, root2, bias2):
    del node_frequency
    x = node_emb[node_index]
    x = jnp.pad(x, ((0, _NP - _N), (0, 0)))
    pad = _EP - _E
    pad_iota = jnp.arange(pad, dtype=jnp.int32)
    srcs = jnp.concatenate(
        [edge_index[0].astype(jnp.int32), pad_iota % _N]
    ).reshape(_CR, _CC)
    dsts = jnp.concatenate(
        [edge_index[1].astype(jnp.int32), _N + pad_iota % (_NP - _N)]
    ).reshape(_CR, _CC)
    rels = jnp.concatenate(
        [edge_type.astype(jnp.int32), jnp.zeros((pad,), jnp.int32)]
    ).reshape(_CR, _CC)
    zero_b = jnp.zeros((_CH, _D), jnp.float32)
    zero_i = jnp.zeros((_NRP,), jnp.int32)
    ones_i = jnp.ones((_CC,), jnp.int32)
    cnt, g2d, nk2d = _sc_count(srcs, dsts, rels, zero_i, ones_i)
    g_flat = g2d.reshape(_EP)
    nk_flat = nk2d.reshape(_EP)
    dst_flat = dsts.reshape(_EP)
    w1 = _wmix(comp1, bases1.reshape(_NB, _D * _D)).reshape(_R, _D, _D)
    w2 = _wmix(comp2, bases2.reshape(_NB, _D * _D)).reshape(_R, _D, _D)
    inv_flat = _invk(cnt.reshape(_NC, _CNT_ROWS, _D)).reshape(_NRP)

    xw1, rootx1 = _dense1(x, w1, root1, bias1.reshape(1, _D))
    acc1 = _sc_layer(g_flat, nk_flat, dst_flat, xw1.reshape(_R * _NP, _D),
                     inv_flat, zero_b)
    xw2, rootx2 = _dense2(acc1, rootx1, w2, root2, bias2.reshape(1, _D))
    acc2 = _sc_layer(g_flat, nk_flat, dst_flat, xw2.reshape(_R * _NP, _D),
                     inv_flat, zero_b)
    return _final(acc2, rootx2)[:_N]


# speedup vs baseline: 2.6127x; 1.0007x over previous
import functools

import jax
import jax.numpy as jnp
from jax import lax
from jax.experimental import pallas as pl
from jax.experimental.pallas import tpu as pltpu
from jax.experimental.pallas import tpu_sc as plsc

_N = 10000
_NP = 10240
_E = 320000
_R = 8
_D = 128
_NB = 34
_NRP = _NP * _R

_NC = 2
_NS = 16
_NW = _NC * _NS
_EP = 327680
_EPW = _EP // _NW

_CC = 128
_CR = _EP // _CC
_CRW = _CR // _NW

_CH = 64
_RPW = _EPW // _CH
_RPS = _NP // _NS
_CPS = _NRP // _NS
_CNT_ROWS = _NRP // _D

_sc_mesh = plsc.VectorSubcoreMesh(core_axis_name="c", subcore_axis_name="s")


@functools.partial(
    pl.kernel,
    out_type=[
        jax.ShapeDtypeStruct((_NC * _NRP,), jnp.int32),
        jax.ShapeDtypeStruct((_CR, _CC), jnp.int32),
        jax.ShapeDtypeStruct((_CR, _CC), jnp.int32),
    ],
    mesh=_sc_mesh,
    scratch_types=[
        pltpu.VMEM_SHARED((_NRP,), jnp.int32),
        pltpu.VMEM((_CRW, _CC), jnp.int32),
        pltpu.VMEM((_CRW, _CC), jnp.int32),
        pltpu.VMEM((_CRW, _CC), jnp.int32),
        pltpu.VMEM((_CC,), jnp.int32),
        pltpu.VMEM((_CPS,), jnp.int32),
        pltpu.SemaphoreType.DMA,
    ],
)
def _sc_count(src_hbm, dst_hbm, rel_hbm, zero_hbm, ones_hbm,
              cnt_out, g_out, nk_out,
              cnt_sp, src2, dst2, rel2, onesb, cstage, sem_s):
    c = lax.axis_index("c")
    s = lax.axis_index("s")
    wid = s * _NC + c
    pltpu.sync_copy(zero_hbm.at[pl.ds(s * _CPS, _CPS)], cstage)
    pltpu.sync_copy(cstage, cnt_sp.at[pl.ds(s * _CPS, _CPS)])
    pltpu.sync_copy(ones_hbm, onesb)
    pltpu.sync_copy(src_hbm.at[pl.ds(wid * _CRW, _CRW)], src2)
    pltpu.sync_copy(dst_hbm.at[pl.ds(wid * _CRW, _CRW)], dst2)
    pltpu.sync_copy(rel_hbm.at[pl.ds(wid * _CRW, _CRW)], rel2)

    @pl.loop(0, _CRW)
    def _keys(j):
        for t in range(_CC // 16):
            sl = pl.ds(t * 16, 16)
            rv = rel2[j, sl]
            src2[j, sl] = src2[j, sl] * _R + rv
            rel2[j, sl] = dst2[j, sl] * _R + rv

    plsc.subcore_barrier()

    @pl.loop(0, _CRW)
    def _fire(j):
        pltpu.async_copy(onesb, cnt_sp.at[rel2.at[j]], sem_s, add=True)

        @pl.when(j >= 8)
        def _():
            pltpu.make_async_copy(onesb, cnt_sp.at[rel2.at[j]], sem_s).wait()

    @pl.loop(0, 8)
    def _drain(j):
        pltpu.make_async_copy(onesb, cnt_sp.at[rel2.at[j]], sem_s).wait()

    pltpu.sync_copy(src2, g_out.at[pl.ds(wid * _CRW, _CRW)])
    pltpu.sync_copy(rel2, nk_out.at[pl.ds(wid * _CRW, _CRW)])

    plsc.subcore_barrier()
    pltpu.sync_copy(cnt_sp.at[pl.ds(s * _CPS, _CPS)], cstage)
    pltpu.sync_copy(cstage, cnt_out.at[pl.ds(c * _NRP + s * _CPS, _CPS)])


@functools.partial(
    pl.kernel,
    out_type=jax.ShapeDtypeStruct((_NC, _NP, _D), jnp.float32),
    mesh=_sc_mesh,
    scratch_types=(
        [pltpu.VMEM_SHARED((_NP, _D), jnp.float32)]
        + [pltpu.VMEM((_CH, _D), jnp.float32)] * 4
        + [pltpu.VMEM((_CH,), jnp.int32)] * 2
        + [pltpu.VMEM((_CH,), jnp.int32)] * 2
        + [pltpu.VMEM((_CH,), jnp.int32)] * 4
        + [pltpu.VMEM((_CH,), jnp.float32)] * 2
        + [pltpu.SemaphoreType.DMA] * 12
    ),
)
def _sc_layer(g_hbm, nk_hbm, dst_hbm, xw_hbm, inv_hbm, zero_hbm, acc_out,
              acc_sp, *bufs):
    rows = bufs[0:4]
    gb = bufs[4:6]
    nkb = bufs[6:8]
    dstb = bufs[8:12]
    invv = bufs[12:14]
    sg = bufs[14:18]
    si = bufs[18:20]
    sx = bufs[20:22]
    ss = bufs[22:26]
    c = lax.axis_index("c")
    s = lax.axis_index("s")
    wid = s * _NC + c
    ebase = wid * _EPW

    pltpu.sync_copy(zero_hbm, rows[0])
    for t in range(_RPS // _CH):
        pltpu.sync_copy(rows[0], acc_sp.at[pl.ds(s * _RPS + t * _CH, _CH)])
    plsc.subcore_barrier()

    def start_idx(j, p, q):
        o = ebase + j * _CH
        pltpu.async_copy(g_hbm.at[pl.ds(o, _CH)], gb[p], sx[p])
        pltpu.async_copy(nk_hbm.at[pl.ds(o, _CH)], nkb[p], sx[p])
        pltpu.async_copy(dst_hbm.at[pl.ds(o, _CH)], dstb[q], sx[p])

    def wait_idx(j, p, q):
        o = ebase + j * _CH
        pltpu.make_async_copy(g_hbm.at[pl.ds(o, _CH)], gb[p], sx[p]).wait()
        pltpu.make_async_copy(nk_hbm.at[pl.ds(o, _CH)], nkb[p], sx[p]).wait()
        pltpu.make_async_copy(
            dst_hbm.at[pl.ds(o, _CH)], dstb[q], sx[p]).wait()

    def start_g(b, p):
        pltpu.async_copy(xw_hbm.at[gb[p]], rows[b], sg[b])
        pltpu.async_copy(inv_hbm.at[nkb[p]], invv[p], si[p])

    def wait_g(b, p):
        pltpu.make_async_copy(xw_hbm.at[gb[p]], rows[b], sg[b]).wait()
        pltpu.make_async_copy(inv_hbm.at[nkb[p]], invv[p], si[p]).wait()

    def start_s(b, q):
        pltpu.async_copy(rows[b], acc_sp.at[dstb[q]], ss[b], add=True)

    def wait_s(b, q):
        pltpu.make_async_copy(rows[b], acc_sp.at[dstb[q]], ss[b]).wait()

    def scale(b, p):
        @pl.loop(0, _CH // 16)
        def _scale(i16):
            i0 = i16 * 16
            iv = invv[p][pl.ds(i0, 16)]
            for t in range(16):
                sv = jnp.full((16,), iv[t])
                for u in range(_D // 16):
                    sl = pl.ds(u * 16, 16)
                    rows[b][i0 + t, sl] = rows[b][i0 + t, sl] * sv

    start_idx(0, 0, 0)
    start_idx(1, 1, 1)
    wait_idx(0, 0, 0)
    start_g(0, 0)
    for j in range(2):
        wait_g(j % 4, j % 2)
        wait_idx(j + 1, (j + 1) % 2, (j + 1) % 4)
        start_g((j + 1) % 4, (j + 1) % 2)
        start_idx(j + 2, (j + 2) % 2, (j + 2) % 4)
        scale(j % 4, j % 2)
        start_s(j % 4, j % 4)

    @pl.loop(2, _RPW + 2, step=4)
    def _main(jj):
        for t in range(4):
            j = jj + t
            p = t % 2
            b = (2 + t) % 4

            @pl.when(j <= _RPW - 1)
            def _():
                wait_g(b, p)
                wait_s((b + 2) % 4, (b + 2) % 4)

                @pl.when(j <= _RPW - 2)
                def _():
                    wait_idx(j + 1, (p + 1) % 2, (b + 1) % 4)
                    start_g((b + 1) % 4, (p + 1) % 2)

                @pl.when(j <= _RPW - 3)
                def _():
                    start_idx(j + 2, p, (b + 2) % 4)

                scale(b, p)
                start_s(b, b)

    wait_s((_RPW - 2) % 4, (_RPW - 2) % 4)
    wait_s((_RPW - 1) % 4, (_RPW - 1) % 4)

    plsc.subcore_barrier()
    for t in range(_RPS // _CH):
        pltpu.sync_copy(acc_sp.at[pl.ds(s * _RPS + t * _CH, _CH)], rows[0])
        pltpu.sync_copy(rows[0],
                        acc_out.at[c, pl.ds(s * _RPS + t * _CH, _CH)])


def _wmix_body(comp_ref, bases_ref, w_ref):
    w_ref[...] = jnp.dot(comp_ref[...], bases_ref[...],
                         preferred_element_type=jnp.float32)


def _wmix(comp, bases2d):
    return pl.pallas_call(
        _wmix_body,
        out_shape=jax.ShapeDtypeStruct((_R, _D * _D), jnp.float32),
    )(comp, bases2d)


def _inv_body(cnt_ref, inv_ref):
    ctot = cnt_ref[0] + cnt_ref[1]
    inv_ref[...] = 1.0 / jnp.maximum(ctot, 1).astype(jnp.float32)


def _invk(cnt):
    return pl.pallas_call(
        _inv_body,
        out_shape=jax.ShapeDtypeStruct((_CNT_ROWS, _D), jnp.float32),
    )(cnt)


_BN = 640
_NBLK = _NP // _BN


def _dense1_body(x_ref, w_ref, root_ref, bias_ref, xw_ref, rootx_ref):
    xb = x_ref[...]
    y = jnp.dot(xb, w_ref[...], preferred_element_type=jnp.float32)
    xw_ref[...] = y.reshape(_BN, _R, _D)
    rootx_ref[...] = (jnp.dot(xb, root_ref[...],
                              preferred_element_type=jnp.float32)
                      + bias_ref[...])


def _dense1(x, w, root, bias):
    return pl.pallas_call(
        _dense1_body,
        grid=(_NBLK,),
        in_specs=[
            pl.BlockSpec((_BN, _D), lambda i: (i, 0)),
            pl.BlockSpec((_D, _R * _D), lambda i: (0, 0)),
            pl.BlockSpec((_D, _D), lambda i: (0, 0)),
            pl.BlockSpec((1, _D), lambda i: (0, 0)),
        ],
        out_specs=[
            pl.BlockSpec((_BN, _R, _D), lambda i: (i, 0, 0)),
            pl.BlockSpec((_BN, _D), lambda i: (i, 0)),
        ],
        out_shape=[
            jax.ShapeDtypeStruct((_NP, _R, _D), jnp.float32),
            jax.ShapeDtypeStruct((_NP, _D), jnp.float32),
        ],
    )(x, w, root, bias)


def _dense2_body(acc_ref, rootx1_ref, w_ref, root_ref, bias_ref,
                 xw_ref, rootx_ref):
    xb = acc_ref[0] + acc_ref[1] + rootx1_ref[...]
    y = jnp.dot(xb, w_ref[...], preferred_element_type=jnp.float32)
    xw_ref[...] = y.reshape(_BN, _R, _D)
    rootx_ref[...] = (jnp.dot(xb, root_ref[...],
                              preferred_element_type=jnp.float32)
                      + bias_ref[...])


def _dense2(acc, rootx1, w, root, bias):
    return pl.pallas_call(
        _dense2_body,
        grid=(_NBLK,),
        in_specs=[
            pl.BlockSpec((_NC, _BN, _D), lambda i: (0, i, 0)),
            pl.BlockSpec((_BN, _D), lambda i: (i, 0)),
            pl.BlockSpec((_D, _R * _D), lambda i: (0, 0)),
            pl.BlockSpec((_D, _D), lambda i: (0, 0)),
            pl.BlockSpec((1, _D), lambda i: (0, 0)),
        ],
        out_specs=[
            pl.BlockSpec((_BN, _R, _D), lambda i: (i, 0, 0)),
            pl.BlockSpec((_BN, _D), lambda i: (i, 0)),
        ],
        out_shape=[
            jax.ShapeDtypeStruct((_NP, _R, _D), jnp.float32),
            jax.ShapeDtypeStruct((_NP, _D), jnp.float32),
        ],
    )(acc, rootx1, w, root, bias)


def _final_body(acc_ref, rootx_ref, out_ref):
    out_ref[...] = acc_ref[0] + acc_ref[1] + rootx_ref[...]


def _final(acc, rootx):
    return pl.pallas_call(
        _final_body,
        grid=(_NBLK,),
        in_specs=[
            pl.BlockSpec((_NC, _BN, _D), lambda i: (0, i, 0)),
            pl.BlockSpec((_BN, _D), lambda i: (i, 0)),
        ],
        out_specs=pl.BlockSpec((_BN, _D), lambda i: (i, 0)),
        out_shape=jax.ShapeDtypeStruct((_NP, _D), jnp.float32),
    )(acc, rootx)


def kernel(node_index, edge_index, edge_type, node_frequency, node_emb,
           comp1, bases1, root1, bias1, comp2, bases2, root2, bias2):
    del node_frequency
    x = node_emb[node_index]
    x = jnp.pad(x, ((0, _NP - _N), (0, 0)))
    pad = _EP - _E
    pad_iota = jnp.arange(pad, dtype=jnp.int32)
    srcs = jnp.concatenate(
        [edge_index[0].astype(jnp.int32), pad_iota % _N]
    ).reshape(_CR, _CC)
    dsts = jnp.concatenate(
        [edge_index[1].astype(jnp.int32), _N + pad_iota % (_NP - _N)]
    ).reshape(_CR, _CC)
    rels = jnp.concatenate(
        [edge_type.astype(jnp.int32), jnp.zeros((pad,), jnp.int32)]
    ).reshape(_CR, _CC)
    zero_b = jnp.zeros((_CH, _D), jnp.float32)
    zero_i = jnp.zeros((_NRP,), jnp.int32)
    ones_i = jnp.ones((_CC,), jnp.int32)
    cnt, g2d, nk2d = _sc_count(srcs, dsts, rels, zero_i, ones_i)
    g_flat = g2d.reshape(_EP)
    nk_flat = nk2d.reshape(_EP)
    dst_flat = dsts.reshape(_EP)
    w1 = (_wmix(comp1, bases1.reshape(_NB, _D * _D))
          .reshape(_R, _D, _D).transpose(1, 0, 2).reshape(_D, _R * _D))
    w2 = (_wmix(comp2, bases2.reshape(_NB, _D * _D))
          .reshape(_R, _D, _D).transpose(1, 0, 2).reshape(_D, _R * _D))
    inv_flat = _invk(cnt.reshape(_NC, _CNT_ROWS, _D)).reshape(_NRP)

    xw1, rootx1 = _dense1(x, w1, root1, bias1.reshape(1, _D))
    acc1 = _sc_layer(g_flat, nk_flat, dst_flat, xw1.reshape(_NP * _R, _D),
                     inv_flat, zero_b)
    xw2, rootx2 = _dense2(acc1, rootx1, w2, root2, bias2.reshape(1, _D))
    acc2 = _sc_layer(g_flat, nk_flat, dst_flat, xw2.reshape(_NP * _R, _D),
                     inv_flat, zero_b)
    return _final(acc2, rootx2)[:_N]
